# baseline (speedup 1.0000x reference)
import jax
import jax.numpy as jnp
from jax import lax
from jax.experimental import pallas as pl
from jax.experimental.pallas import tpu as pltpu
from jax.experimental.pallas import tpu_sc as plsc

NUM_MOVIES = 1000000
FEAT_DIM = 64
EMBED_DIM = 32
BATCH = 16384

NC = 2
NS = 16
NW = NC * NS
B_PER_W = BATCH // NW
GRP = 8
OUT_W = 128


def _gather_body(idx_hbm, tab_hbm, out_hbm, idx_v, buf_a, buf_b, out_v,
                 sem_a, sem_b):
    wid = lax.axis_index("s") * NC + lax.axis_index("c")
    base = wid * B_PER_W
    pltpu.sync_copy(idx_hbm.at[pl.ds(base, B_PER_W)], idx_v)

    def fire(buf, sem, v16, lane0):
        for j in range(GRP):
            row0 = pl.multiple_of(
                lax.bitwise_and(v16[lane0 + j], jnp.int32(~7)), 8)
            pltpu.async_copy(tab_hbm.at[pl.ds(row0, 8)],
                             buf.at[pl.ds(j * 8, 8)], sem)

    def drain(buf, sem):
        for j in range(GRP):
            pltpu.make_async_copy(tab_hbm.at[pl.ds(0, 8)],
                                  buf.at[pl.ds(j * 8, 8)], sem).wait()

    def select(buf, v16, lane0, r0):
        for j in range(GRP):
            sub = lax.bitwise_and(v16[lane0 + j], jnp.int32(7))
            out_v[r0 + j, pl.ds(0, 16)] = buf[j * 8 + sub, pl.ds(0, 16)]
            out_v[r0 + j, pl.ds(16, 16)] = buf[j * 8 + sub, pl.ds(16, 16)]

    v16_0 = idx_v[pl.ds(0, 16)]
    fire(buf_a, sem_a, v16_0, 0)
    fire(buf_b, sem_b, v16_0, GRP)
    n_iter = B_PER_W // (2 * GRP)

    def body(t, _):
        r0 = t * (2 * GRP)
        v16 = idx_v[pl.ds(r0, 16)]
        drain(buf_a, sem_a)
        select(buf_a, v16, 0, r0)

        @pl.when(t < n_iter - 1)
        def _():
            nv16 = idx_v[pl.ds(r0 + 16, 16)]
            fire(buf_a, sem_a, nv16, 0)

        drain(buf_b, sem_b)
        select(buf_b, v16, GRP, r0 + GRP)

        @pl.when(t < n_iter - 1)
        def _():
            nv16 = idx_v[pl.ds(r0 + 16, 16)]
            fire(buf_b, sem_b, nv16, GRP)

        return 0

    lax.fori_loop(0, n_iter, body, 0)
    pltpu.sync_copy(out_v, out_hbm.at[pl.ds(base, B_PER_W)])


@jax.jit
def _sc_gather(movie_id, tab):
    mesh = plsc.VectorSubcoreMesh(core_axis_name="c", subcore_axis_name="s")
    return pl.kernel(
        _gather_body,
        mesh=mesh,
        out_type=jax.ShapeDtypeStruct((BATCH, OUT_W), jnp.float32),
        scratch_types=[
            pltpu.VMEM((B_PER_W,), jnp.int32),
            pltpu.VMEM((GRP * 8, EMBED_DIM), jnp.float32),
            pltpu.VMEM((GRP * 8, EMBED_DIM), jnp.float32),
            pltpu.VMEM((B_PER_W, OUT_W), jnp.float32),
            pltpu.SemaphoreType.DMA,
            pltpu.SemaphoreType.DMA,
        ],
        compiler_params=pltpu.CompilerParams(use_tc_tiling_on_sc=True),
    )(movie_id, tab)


BLK = 2048


def _mlp_body(e_ref, f_ref, w1e_ref, w1f_ref, b1_ref, w2_ref, b2_ref,
              w3_ref, b3_ref, o_ref):
    e = e_ref[...][:, :EMBED_DIM]
    h = jnp.dot(e, w1e_ref[...], preferred_element_type=jnp.float32)
    h = h + jnp.dot(f_ref[...], w1f_ref[...],
                    preferred_element_type=jnp.float32)
    h = jnp.maximum(h + b1_ref[...], 0.0)
    h = jnp.maximum(
        jnp.dot(h, w2_ref[...], preferred_element_type=jnp.float32)
        + b2_ref[...], 0.0)
    o_ref[...] = (
        jnp.dot(h, w3_ref[...], preferred_element_type=jnp.float32)
        + b3_ref[...])


def _full(shape):
    return pl.BlockSpec(shape, lambda i: (0, 0))


@jax.jit
def _tc_mlp(embed, feat, w1e_t, w1f_t, b1, w2_t, b2, w3_t, b3):
    grid = (BATCH // BLK,)
    return pl.pallas_call(
        _mlp_body,
        grid=grid,
        in_specs=[
            pl.BlockSpec((BLK, OUT_W), lambda i: (i, 0)),
            pl.BlockSpec((BLK, FEAT_DIM), lambda i: (i, 0)),
            _full(w1e_t.shape),
            _full(w1f_t.shape),
            _full(b1.shape),
            _full(w2_t.shape),
            _full(b2.shape),
            _full(w3_t.shape),
            _full(b3.shape),
        ],
        out_specs=pl.BlockSpec((BLK, EMBED_DIM), lambda i: (i, 0)),
        out_shape=jax.ShapeDtypeStruct((BATCH, EMBED_DIM), jnp.float32),
    )(embed, feat, w1e_t, w1f_t, b1, w2_t, b2, w3_t, b3)


def kernel(movie_id, movie_features, table, W1, b1, W2, b2, W3, b3):
    embed = _sc_gather(movie_id, table)
    w1e_t = W1[:, :EMBED_DIM].T
    w1f_t = W1[:, EMBED_DIM:].T
    return _tc_mlp(embed, movie_features, w1e_t, w1f_t, b1.reshape(1, -1),
                   W2.T, b2.reshape(1, -1), W3.T, b3.reshape(1, -1))


# speedup vs baseline: 2.2420x; 2.2420x over previous
import jax
import jax.numpy as jnp
from jax import lax
from jax.experimental import pallas as pl
from jax.experimental.pallas import tpu as pltpu
from jax.experimental.pallas import tpu_sc as plsc

NUM_MOVIES = 1000000
FEAT_DIM = 64
EMBED_DIM = 32
BATCH = 16384

NC = 2
NS = 16
NW = NC * NS
B_PER_W = BATCH // NW
GRP = 8
OUT_W = 128


def _gather_body(idx_hbm, tabt_hbm, out_hbm, idx_v, buf_a, buf_b, out_v,
                 sem_a, sem_b):
    wid = lax.axis_index("s") * NC + lax.axis_index("c")
    base = wid * B_PER_W
    pltpu.sync_copy(idx_hbm.at[pl.ds(base, B_PER_W)], idx_v)
    iota = lax.iota(jnp.int32, 16)

    def fire(buf, sem, v16, lane0):
        for j in range(GRP):
            col0 = pl.multiple_of(
                lax.bitwise_and(v16[lane0 + j], jnp.int32(~127)), 128)
            pltpu.async_copy(tabt_hbm.at[:, pl.ds(col0, 128)],
                             buf.at[pl.ds(j * EMBED_DIM, EMBED_DIM)], sem)

    def drain(buf, sem):
        for j in range(GRP):
            pltpu.make_async_copy(tabt_hbm.at[:, pl.ds(0, 128)],
                                  buf.at[pl.ds(j * EMBED_DIM, EMBED_DIM)],
                                  sem).wait()

    def select(buf, v16, lane0, r0):
        for j in range(GRP):
            p16 = jnp.full((16,), lax.bitwise_and(v16[lane0 + j],
                                                  jnp.int32(127)), jnp.int32)
            lo = plsc.load_gather(buf, [j * EMBED_DIM + iota, p16])
            hi = plsc.load_gather(buf, [j * EMBED_DIM + 16 + iota, p16])
            r = r0 + j
            row = lax.shift_right_logical(r, 2)
            cb = lax.bitwise_and(r, jnp.int32(3)) * EMBED_DIM
            out_v[row, pl.ds(cb, 16)] = lo
            out_v[row, pl.ds(cb + 16, 16)] = hi

    v16_0 = idx_v[pl.ds(0, 16)]
    fire(buf_a, sem_a, v16_0, 0)
    fire(buf_b, sem_b, v16_0, GRP)
    n_iter = B_PER_W // (2 * GRP)

    def body(t, _):
        r0 = t * (2 * GRP)
        v16 = idx_v[pl.ds(r0, 16)]
        drain(buf_a, sem_a)
        select(buf_a, v16, 0, r0)

        @pl.when(t < n_iter - 1)
        def _():
            nv16 = idx_v[pl.ds(r0 + 16, 16)]
            fire(buf_a, sem_a, nv16, 0)

        drain(buf_b, sem_b)
        select(buf_b, v16, GRP, r0 + GRP)

        @pl.when(t < n_iter - 1)
        def _():
            nv16 = idx_v[pl.ds(r0 + 16, 16)]
            fire(buf_b, sem_b, nv16, GRP)

        return 0

    lax.fori_loop(0, n_iter, body, 0)
    pltpu.sync_copy(out_v, out_hbm.at[pl.ds(wid * (B_PER_W // 4), B_PER_W // 4)])


@jax.jit
def _sc_gather(movie_id, tabt):
    mesh = plsc.VectorSubcoreMesh(core_axis_name="c", subcore_axis_name="s")
    return pl.kernel(
        _gather_body,
        mesh=mesh,
        out_type=jax.ShapeDtypeStruct((BATCH * EMBED_DIM // 128, 128),
                                      jnp.float32),
        scratch_types=[
            pltpu.VMEM((B_PER_W,), jnp.int32),
            pltpu.VMEM((GRP * EMBED_DIM, 128), jnp.float32),
            pltpu.VMEM((GRP * EMBED_DIM, 128), jnp.float32),
            pltpu.VMEM((B_PER_W // 4, 128), jnp.float32),
            pltpu.SemaphoreType.DMA,
            pltpu.SemaphoreType.DMA,
        ],
        compiler_params=pltpu.CompilerParams(use_tc_tiling_on_sc=True,
                                             needs_layout_passes=False),
    )(movie_id, tabt)


BLK = 2048


def _mlp_body(e_ref, f_ref, w1e_ref, w1f_ref, b1_ref, w2_ref, b2_ref,
              w3_ref, b3_ref, o_ref):
    h = jnp.dot(e_ref[...], w1e_ref[...], preferred_element_type=jnp.float32)
    h = h + jnp.dot(f_ref[...], w1f_ref[...],
                    preferred_element_type=jnp.float32)
    h = jnp.maximum(h + b1_ref[...], 0.0)
    h = jnp.maximum(
        jnp.dot(h, w2_ref[...], preferred_element_type=jnp.float32)
        + b2_ref[...], 0.0)
    o_ref[...] = (
        jnp.dot(h, w3_ref[...], preferred_element_type=jnp.float32)
        + b3_ref[...])


def _full(shape):
    return pl.BlockSpec(shape, lambda i: (0, 0))


@jax.jit
def _tc_mlp(embed, feat, w1e_t, w1f_t, b1, w2_t, b2, w3_t, b3):
    grid = (BATCH // BLK,)
    return pl.pallas_call(
        _mlp_body,
        grid=grid,
        in_specs=[
            pl.BlockSpec((BLK, EMBED_DIM), lambda i: (i, 0)),
            pl.BlockSpec((BLK, FEAT_DIM), lambda i: (i, 0)),
            _full(w1e_t.shape),
            _full(w1f_t.shape),
            _full(b1.shape),
            _full(w2_t.shape),
            _full(b2.shape),
            _full(w3_t.shape),
            _full(b3.shape),
        ],
        out_specs=pl.BlockSpec((BLK, EMBED_DIM), lambda i: (i, 0)),
        out_shape=jax.ShapeDtypeStruct((BATCH, EMBED_DIM), jnp.float32),
    )(embed, feat, w1e_t, w1f_t, b1, w2_t, b2, w3_t, b3)


def kernel(movie_id, movie_features, table, W1, b1, W2, b2, W3, b3):
    embed = _sc_gather(movie_id, table.T).reshape(BATCH, EMBED_DIM)
    w1e_t = W1[:, :EMBED_DIM].T
    w1f_t = W1[:, EMBED_DIM:].T
    return _tc_mlp(embed, movie_features, w1e_t, w1f_t, b1.reshape(1, -1),
                   W2.T, b2.reshape(1, -1), W3.T, b3.reshape(1, -1))


# speedup vs baseline: 2.2847x; 1.0191x over previous
import jax
import jax.numpy as jnp
from jax import lax
from jax.experimental import pallas as pl
from jax.experimental.pallas import tpu as pltpu
from jax.experimental.pallas import tpu_sc as plsc

NUM_MOVIES = 1000000
FEAT_DIM = 64
EMBED_DIM = 32
BATCH = 16384

NC = 2
NS = 16
NW = NC * NS
B_PER_W = BATCH // NW
GRP = 8
OUT_W = 128


def _gather_body(b_per_w, idx_hbm, tabt_hbm, out_hbm, idx_v, buf_a, buf_b,
                 out_v, sem_a, sem_b):
    wid = lax.axis_index("s") * NC + lax.axis_index("c")
    base = wid * b_per_w
    pltpu.sync_copy(idx_hbm.at[pl.ds(base, b_per_w)], idx_v)
    iota = lax.iota(jnp.int32, 16)

    def fire(buf, sem, v16, lane0):
        for j in range(GRP):
            col0 = pl.multiple_of(
                lax.bitwise_and(v16[lane0 + j], jnp.int32(~127)), 128)
            pltpu.async_copy(tabt_hbm.at[:, pl.ds(col0, 128)],
                             buf.at[pl.ds(j * EMBED_DIM, EMBED_DIM)], sem)

    def drain(buf, sem):
        for j in range(GRP):
            pltpu.make_async_copy(tabt_hbm.at[:, pl.ds(0, 128)],
                                  buf.at[pl.ds(j * EMBED_DIM, EMBED_DIM)],
                                  sem).wait()

    def select(buf, v16, lane0, r0):
        for j in range(GRP):
            p16 = jnp.full((16,), lax.bitwise_and(v16[lane0 + j],
                                                  jnp.int32(127)), jnp.int32)
            lo = plsc.load_gather(buf, [j * EMBED_DIM + iota, p16])
            hi = plsc.load_gather(buf, [j * EMBED_DIM + 16 + iota, p16])
            r = r0 + j
            row = lax.shift_right_logical(r, 2)
            cb = lax.bitwise_and(r, jnp.int32(3)) * EMBED_DIM
            out_v[row, pl.ds(cb, 16)] = lo
            out_v[row, pl.ds(cb + 16, 16)] = hi

    v16_0 = idx_v[pl.ds(0, 16)]
    fire(buf_a, sem_a, v16_0, 0)
    fire(buf_b, sem_b, v16_0, GRP)
    n_iter = b_per_w // (2 * GRP)

    def body(t, _):
        r0 = t * (2 * GRP)
        v16 = idx_v[pl.ds(r0, 16)]
        drain(buf_a, sem_a)
        select(buf_a, v16, 0, r0)

        @pl.when(t < n_iter - 1)
        def _():
            nv16 = idx_v[pl.ds(r0 + 16, 16)]
            fire(buf_a, sem_a, nv16, 0)

        drain(buf_b, sem_b)
        select(buf_b, v16, GRP, r0 + GRP)

        @pl.when(t < n_iter - 1)
        def _():
            nv16 = idx_v[pl.ds(r0 + 16, 16)]
            fire(buf_b, sem_b, nv16, GRP)

        return 0

    lax.fori_loop(0, n_iter, body, 0)
    pltpu.sync_copy(out_v, out_hbm.at[pl.ds(wid * (b_per_w // 4), b_per_w // 4)])


def _sc_gather(movie_id, tabt):
    import functools
    n = movie_id.shape[0]
    b_per_w = n // NW
    mesh = plsc.VectorSubcoreMesh(core_axis_name="c", subcore_axis_name="s")
    return pl.kernel(
        functools.partial(_gather_body, b_per_w),
        mesh=mesh,
        out_type=jax.ShapeDtypeStruct((n * EMBED_DIM // 128, 128),
                                      jnp.float32),
        scratch_types=[
            pltpu.VMEM((b_per_w,), jnp.int32),
            pltpu.VMEM((GRP * EMBED_DIM, 128), jnp.float32),
            pltpu.VMEM((GRP * EMBED_DIM, 128), jnp.float32),
            pltpu.VMEM((b_per_w // 4, 128), jnp.float32),
            pltpu.SemaphoreType.DMA,
            pltpu.SemaphoreType.DMA,
        ],
        compiler_params=pltpu.CompilerParams(use_tc_tiling_on_sc=True,
                                             needs_layout_passes=False),
    )(movie_id, tabt)


BLK = 2048


def _mlp_body(e_ref, f_ref, w1e_ref, w1f_ref, b1_ref, w2_ref, b2_ref,
              w3_ref, b3_ref, o_ref):
    h = jnp.dot(e_ref[...], w1e_ref[...], preferred_element_type=jnp.float32)
    h = h + jnp.dot(f_ref[...], w1f_ref[...],
                    preferred_element_type=jnp.float32)
    h = jnp.maximum(h + b1_ref[...], 0.0)
    h = jnp.maximum(
        jnp.dot(h, w2_ref[...], preferred_element_type=jnp.float32)
        + b2_ref[...], 0.0)
    o_ref[...] = (
        jnp.dot(h, w3_ref[...], preferred_element_type=jnp.float32)
        + b3_ref[...])


def _full(shape):
    return pl.BlockSpec(shape, lambda i: (0, 0))


def _tc_mlp(embed, feat, w1e_t, w1f_t, b1, w2_t, b2, w3_t, b3):
    n = embed.shape[0]
    grid = (n // BLK,)
    return pl.pallas_call(
        _mlp_body,
        grid=grid,
        in_specs=[
            pl.BlockSpec((BLK, EMBED_DIM), lambda i: (i, 0)),
            pl.BlockSpec((BLK, FEAT_DIM), lambda i: (i, 0)),
            _full(w1e_t.shape),
            _full(w1f_t.shape),
            _full(b1.shape),
            _full(w2_t.shape),
            _full(b2.shape),
            _full(w3_t.shape),
            _full(b3.shape),
        ],
        out_specs=pl.BlockSpec((BLK, EMBED_DIM), lambda i: (i, 0)),
        out_shape=jax.ShapeDtypeStruct((n, EMBED_DIM), jnp.float32),
    )(embed, feat, w1e_t, w1f_t, b1, w2_t, b2, w3_t, b3)


def kernel(movie_id, movie_features, table, W1, b1, W2, b2, W3, b3):
    tabt = table.T
    w1e_t = W1[:, :EMBED_DIM].T
    w1f_t = W1[:, EMBED_DIM:].T
    b1r, b2r, b3r = b1.reshape(1, -1), b2.reshape(1, -1), b3.reshape(1, -1)
    w2_t, w3_t = W2.T, W3.T
    half = BATCH // 2
    g0 = _sc_gather(movie_id[:half], tabt)
    g1 = _sc_gather(movie_id[half:], tabt)
    e0 = g0.reshape(half, EMBED_DIM)
    e1 = g1.reshape(half, EMBED_DIM)
    o0 = _tc_mlp(e0, movie_features[:half], w1e_t, w1f_t, b1r, w2_t, b2r,
                 w3_t, b3r)
    o1 = _tc_mlp(e1, movie_features[half:], w1e_t, w1f_t, b1r, w2_t, b2r,
                 w3_t, b3r)
    return jnp.concatenate([o0, o1], axis=0)
